# baseline (speedup 1.0000x reference)
import functools

import jax
import jax.numpy as jnp
from jax import lax
from jax.experimental import pallas as pl
from jax.experimental.pallas import tpu as pltpu
from jax.experimental.pallas import tpu_sc as plsc

_NC = 2
_NS = 16
_NW = _NC * _NS
_L = 16


def _make_lookup(n_pts, d0, d1, d2, chunk):
    n_words = (d0 * d1 * d2) // 32
    oob = n_words
    per_w = n_pts // _NW
    n_chunks = per_w // chunk
    mesh = plsc.VectorSubcoreMesh(core_axis_name="c", subcore_axis_name="s")

    @functools.partial(
        pl.kernel,
        mesh=mesh,
        out_type=jax.ShapeDtypeStruct((n_pts,), jnp.int32),
        scratch_types=[
            pltpu.VMEM((3 * chunk,), jnp.float32),
            pltpu.VMEM((chunk,), jnp.int32),
            pltpu.VMEM((chunk,), jnp.int32),
            pltpu.VMEM((chunk,), jnp.int32),
            pltpu.VMEM((chunk,), jnp.int32),
            pltpu.VMEM((6 * _L,), jnp.float32),
            pltpu.SemaphoreType.DMA,
        ],
        compiler_params=pltpu.CompilerParams(needs_layout_passes=False),
    )
    def lookup(xyz_hbm, tab_hbm, c6_hbm, out_hbm,
               xyz_v, idx_v, bit_v, word_v, out_v, c6_v, sem):
        wid = lax.axis_index("s") * _NC + lax.axis_index("c")
        pltpu.sync_copy(c6_hbm, c6_v)
        sx = c6_v[pl.ds(0 * _L, _L)]
        sy = c6_v[pl.ds(1 * _L, _L)]
        sz = c6_v[pl.ds(2 * _L, _L)]
        tx = c6_v[pl.ds(3 * _L, _L)]
        ty = c6_v[pl.ds(4 * _L, _L)]
        tz = c6_v[pl.ds(5 * _L, _L)]
        lane3 = lax.iota(jnp.int32, _L) * 3
        hi_x = jnp.float32(d0) - 0.5
        hi_y = jnp.float32(d1) - 0.5
        hi_z = jnp.float32(d2) - 0.5

        def rnd(v, hi):
            c = jnp.clip(v, 0.0, hi - 0.5)
            f = (c + 0.5).astype(jnp.int32)
            tie = (f.astype(jnp.float32) - c) == 0.5
            return f - jnp.where(tie, f & 1, 0)

        def chunk_body(ci, base):
            base = pl.multiple_of(base, chunk)
            pltpu.sync_copy(xyz_hbm.at[pl.ds(3 * base, 3 * chunk)], xyz_v)

            def pt(t, _):
                p3 = t * (3 * _L) + lane3
                x = plsc.load_gather(xyz_v, [p3])
                y = plsc.load_gather(xyz_v, [p3 + 1])
                z = plsc.load_gather(xyz_v, [p3 + 2])
                vx = x * sx + tx
                vy = y * sy + ty
                vz = z * sz + tz
                inb = ((vx >= -0.5) & (vx < hi_x)
                       & (vy >= -0.5) & (vy < hi_y)
                       & (vz >= -0.5) & (vz < hi_z))
                fx = rnd(vx, hi_x)
                fy = rnd(vy, hi_y)
                fz = rnd(vz, hi_z)
                lin = (fx * (d1 * d2) + fy * d2) + fz
                s = pl.ds(t * _L, _L)
                idx_v[s] = jnp.where(inb, lax.shift_right_logical(lin, 5), oob)
                bit_v[s] = lin & 31
                return 0

            lax.fori_loop(0, chunk // _L, pt, 0)
            pltpu.async_copy(tab_hbm.at[idx_v], word_v, sem).wait()

            def ob(t, _):
                s = pl.ds(t * _L, _L)
                out_v[s] = lax.shift_right_logical(word_v[s], bit_v[s]) & 1
                return 0

            lax.fori_loop(0, chunk // _L, ob, 0)
            pltpu.sync_copy(out_v, out_hbm.at[pl.ds(base, chunk)])
            return base + chunk

        lax.fori_loop(0, n_chunks, chunk_body, wid * per_w)

    return lookup


def kernel(xyz, mask, xyz2ijk_scale, xyz2ijk_shift):
    shape = xyz.shape[:-1]
    n_pts = xyz.size // 3
    d0, d1, d2 = mask.shape
    m = mask.reshape(-1, 32).astype(jnp.uint32)
    bits = jnp.arange(32, dtype=jnp.uint32)[None, :]
    words = jnp.sum(m << bits, axis=1, dtype=jnp.uint32)
    tab = jnp.concatenate(
        [lax.bitcast_convert_type(words, jnp.int32),
         jnp.zeros((8,), jnp.int32)])
    c6 = jnp.repeat(
        jnp.concatenate([xyz2ijk_scale, xyz2ijk_shift])[:, None], _L, axis=1
    ).reshape(-1)
    lookup = _make_lookup(n_pts, d0, d1, d2, chunk=8192)
    out = lookup(xyz.reshape(-1), tab, c6)
    return out.astype(jnp.bool_).reshape(shape)


# speedup vs baseline: 1.0326x; 1.0326x over previous
import functools

import jax
import jax.numpy as jnp
from jax import lax
from jax.experimental import pallas as pl
from jax.experimental.pallas import tpu as pltpu
from jax.experimental.pallas import tpu_sc as plsc

_NC = 2
_NS = 16
_NW = _NC * _NS
_L = 16


def _make_lookup(n_pts, d0, d1, d2, chunk):
    n_words = (d0 * d1 * d2) // 32
    oob = n_words
    per_w = n_pts // _NW
    n_chunks = per_w // chunk
    mesh = plsc.VectorSubcoreMesh(core_axis_name="c", subcore_axis_name="s")

    @functools.partial(
        pl.kernel,
        mesh=mesh,
        out_type=jax.ShapeDtypeStruct((n_pts,), jnp.int32),
        scratch_types=[
            pltpu.VMEM((3 * chunk,), jnp.float32),
            pltpu.VMEM((chunk,), jnp.int32),
            pltpu.VMEM((chunk,), jnp.int32),
            pltpu.VMEM((chunk,), jnp.int32),
            pltpu.VMEM((chunk,), jnp.int32),
            pltpu.VMEM((6 * _L,), jnp.float32),
            pltpu.SemaphoreType.DMA,
        ],
        compiler_params=pltpu.CompilerParams(needs_layout_passes=False),
    )
    def lookup(xyz_hbm, tab_hbm, c6_hbm, out_hbm,
               xyz_v, idx_v, bit_v, word_v, out_v, c6_v, sem):
        wid = lax.axis_index("s") * _NC + lax.axis_index("c")
        pltpu.sync_copy(c6_hbm, c6_v)
        sx = c6_v[pl.ds(0 * _L, _L)]
        sy = c6_v[pl.ds(1 * _L, _L)]
        sz = c6_v[pl.ds(2 * _L, _L)]
        tx = c6_v[pl.ds(3 * _L, _L)]
        ty = c6_v[pl.ds(4 * _L, _L)]
        tz = c6_v[pl.ds(5 * _L, _L)]
        lane3 = lax.iota(jnp.int32, _L) * 3
        hi_x = jnp.float32(d0) - 0.5
        hi_y = jnp.float32(d1) - 0.5
        hi_z = jnp.float32(d2) - 0.5

        def rnd(v, hi):
            c = jnp.clip(v, 0.0, hi - 0.5)
            f = (c + 0.5).astype(jnp.int32)
            tie = (f.astype(jnp.float32) - c) == 0.5
            return f - jnp.where(tie, f & 1, 0)

        def chunk_body(ci, base):
            base = pl.multiple_of(base, chunk)
            pltpu.sync_copy(xyz_hbm.at[pl.ds(3 * base, 3 * chunk)], xyz_v)

            def pt(t, _):
                p3 = t * (3 * _L) + lane3
                x = plsc.load_gather(xyz_v, [p3])
                y = plsc.load_gather(xyz_v, [p3 + 1])
                z = plsc.load_gather(xyz_v, [p3 + 2])
                vx = x * sx + tx
                vy = y * sy + ty
                vz = z * sz + tz
                inb = ((vx >= -0.5) & (vx < hi_x)
                       & (vy >= -0.5) & (vy < hi_y)
                       & (vz >= -0.5) & (vz < hi_z))
                fx = rnd(vx, hi_x)
                fy = rnd(vy, hi_y)
                fz = rnd(vz, hi_z)
                wi = (lax.shift_right_logical(fx, 5) * (d1 * d2)
                      + fy * d2) + fz
                s = pl.ds(t * _L, _L)
                idx_v[s] = jnp.where(inb, wi, oob)
                bit_v[s] = fx & 31
                return 0

            lax.fori_loop(0, chunk // _L, pt, 0)
            pltpu.async_copy(tab_hbm.at[idx_v], word_v, sem).wait()

            def ob(t, _):
                s = pl.ds(t * _L, _L)
                out_v[s] = lax.shift_right_logical(word_v[s], bit_v[s]) & 1
                return 0

            lax.fori_loop(0, chunk // _L, ob, 0)
            pltpu.sync_copy(out_v, out_hbm.at[pl.ds(base, chunk)])
            return base + chunk

        lax.fori_loop(0, n_chunks, chunk_body, wid * per_w)

    return lookup


def kernel(xyz, mask, xyz2ijk_scale, xyz2ijk_shift):
    shape = xyz.shape[:-1]
    n_pts = xyz.size // 3
    d0, d1, d2 = mask.shape
    m4 = mask.reshape(d0 // 32, 32, d1, d2).astype(jnp.uint32)
    bits = jnp.arange(32, dtype=jnp.uint32).reshape(1, 32, 1, 1)
    words3 = jnp.sum(m4 << bits, axis=1, dtype=jnp.uint32)
    tab = jnp.concatenate(
        [lax.bitcast_convert_type(words3, jnp.int32).reshape(-1),
         jnp.zeros((8,), jnp.int32)])
    c6 = jnp.repeat(
        jnp.concatenate([xyz2ijk_scale, xyz2ijk_shift])[:, None], _L, axis=1
    ).reshape(-1)
    lookup = _make_lookup(n_pts, d0, d1, d2, chunk=8192)
    out = lookup(xyz.reshape(-1), tab, c6)
    return out.astype(jnp.bool_).reshape(shape)


# speedup vs baseline: 10.9189x; 10.5743x over previous
import functools

import jax
import jax.numpy as jnp
from jax import lax
from jax.experimental import pallas as pl
from jax.experimental.pallas import tpu as pltpu
from jax.experimental.pallas import tpu_sc as plsc

_NC = 2
_NS = 16
_NW = _NC * _NS
_L = 16


def _make_lookup(n_pts, d0, d1, d2, chunk):
    n_words = (d0 * d1 * d2) // 32
    oob = n_words
    per_w = n_pts // _NW
    n_chunks = per_w // chunk
    mesh = plsc.VectorSubcoreMesh(core_axis_name="c", subcore_axis_name="s")

    @functools.partial(
        pl.kernel,
        mesh=mesh,
        out_type=jax.ShapeDtypeStruct((n_pts,), jnp.int32),
        scratch_types=[
            pltpu.VMEM((chunk,), jnp.float32),
            pltpu.VMEM((chunk,), jnp.float32),
            pltpu.VMEM((chunk,), jnp.float32),
            pltpu.VMEM((chunk,), jnp.int32),
            pltpu.VMEM((chunk,), jnp.int32),
            pltpu.VMEM((chunk,), jnp.int32),
            pltpu.VMEM((chunk,), jnp.int32),
            pltpu.VMEM((6 * _L,), jnp.float32),
            pltpu.SemaphoreType.DMA,
        ],
        compiler_params=pltpu.CompilerParams(needs_layout_passes=False),
    )
    def lookup(x_hbm, y_hbm, z_hbm, tab_hbm, c6_hbm, out_hbm,
               x_v, y_v, z_v, idx_v, bit_v, word_v, out_v, c6_v, sem):
        wid = lax.axis_index("s") * _NC + lax.axis_index("c")
        pltpu.sync_copy(c6_hbm, c6_v)
        sx = c6_v[pl.ds(0 * _L, _L)]
        sy = c6_v[pl.ds(1 * _L, _L)]
        sz = c6_v[pl.ds(2 * _L, _L)]
        tx = c6_v[pl.ds(3 * _L, _L)]
        ty = c6_v[pl.ds(4 * _L, _L)]
        tz = c6_v[pl.ds(5 * _L, _L)]
        hi_x = jnp.float32(d0) - 0.5
        hi_y = jnp.float32(d1) - 0.5
        hi_z = jnp.float32(d2) - 0.5

        def rnd(v, hi):
            c = jnp.clip(v, 0.0, hi - 0.5)
            f = (c + 0.5).astype(jnp.int32)
            tie = (f.astype(jnp.float32) - c) == 0.5
            return f - jnp.where(tie, f & 1, 0)

        def chunk_body(ci, base):
            base = pl.multiple_of(base, chunk)
            pltpu.sync_copy(x_hbm.at[pl.ds(base, chunk)], x_v)
            pltpu.sync_copy(y_hbm.at[pl.ds(base, chunk)], y_v)
            pltpu.sync_copy(z_hbm.at[pl.ds(base, chunk)], z_v)

            def pt(t, _):
                s = pl.ds(t * _L, _L)
                vx = x_v[s] * sx + tx
                vy = y_v[s] * sy + ty
                vz = z_v[s] * sz + tz
                inb = ((vx >= -0.5) & (vx < hi_x)
                       & (vy >= -0.5) & (vy < hi_y)
                       & (vz >= -0.5) & (vz < hi_z))
                fx = rnd(vx, hi_x)
                fy = rnd(vy, hi_y)
                fz = rnd(vz, hi_z)
                wi = (lax.shift_right_logical(fx, 5) * (d1 * d2)
                      + fy * d2) + fz
                idx_v[s] = jnp.where(inb, wi, oob)
                bit_v[s] = fx & 31
                return 0

            lax.fori_loop(0, chunk // _L, pt, 0)
            pltpu.async_copy(tab_hbm.at[idx_v], word_v, sem).wait()

            def ob(t, _):
                s = pl.ds(t * _L, _L)
                out_v[s] = lax.shift_right_logical(word_v[s], bit_v[s]) & 1
                return 0

            lax.fori_loop(0, chunk // _L, ob, 0)
            pltpu.sync_copy(out_v, out_hbm.at[pl.ds(base, chunk)])
            return base + chunk

        lax.fori_loop(0, n_chunks, chunk_body, wid * per_w)

    return lookup


def kernel(xyz, mask, xyz2ijk_scale, xyz2ijk_shift):
    shape = xyz.shape[:-1]
    n_pts = xyz.size // 3
    d0, d1, d2 = mask.shape
    m4 = mask.reshape(d0 // 32, 32, d1, d2).astype(jnp.uint32)
    bits = jnp.arange(32, dtype=jnp.uint32).reshape(1, 32, 1, 1)
    words3 = jnp.sum(m4 << bits, axis=1, dtype=jnp.uint32)
    tab = jnp.concatenate(
        [lax.bitcast_convert_type(words3, jnp.int32).reshape(-1),
         jnp.zeros((8,), jnp.int32)])
    xt = xyz.reshape(-1, 3).T
    c6 = jnp.repeat(
        jnp.concatenate([xyz2ijk_scale, xyz2ijk_shift])[:, None], _L, axis=1
    ).reshape(-1)
    lookup = _make_lookup(n_pts, d0, d1, d2, chunk=8192)
    out = lookup(xt[0], xt[1], xt[2], tab, c6)
    return out.astype(jnp.bool_).reshape(shape)


# speedup vs baseline: 13.7022x; 1.2549x over previous
import functools

import jax
import jax.numpy as jnp
from jax import lax
from jax.experimental import pallas as pl
from jax.experimental.pallas import tpu as pltpu
from jax.experimental.pallas import tpu_sc as plsc

_NC = 2
_NS = 16
_NW = _NC * _NS
_L = 16


def _make_lookup(n_pts, d0, d1, d2, chunk):
    n_words = (d0 * d1 * d2) // 32
    oob = n_words
    per_w = n_pts // _NW
    n_chunks = per_w // chunk
    mesh = plsc.VectorSubcoreMesh(core_axis_name="c", subcore_axis_name="s")

    @functools.partial(
        pl.kernel,
        mesh=mesh,
        out_type=jax.ShapeDtypeStruct((n_pts,), jnp.int32),
        scratch_types=[
            pltpu.VMEM((chunk,), jnp.float32),
            pltpu.VMEM((chunk,), jnp.float32),
            pltpu.VMEM((chunk,), jnp.float32),
            pltpu.VMEM((chunk,), jnp.int32),
            pltpu.VMEM((chunk,), jnp.int32),
            pltpu.VMEM((chunk,), jnp.int32),
            pltpu.VMEM((chunk,), jnp.int32),
            pltpu.VMEM((6 * _L,), jnp.float32),
            pltpu.VMEM_SHARED((n_words + 8,), jnp.int32),
            pltpu.SemaphoreType.DMA,
        ],
        compiler_params=pltpu.CompilerParams(needs_layout_passes=False),
    )
    def lookup(x_hbm, y_hbm, z_hbm, tab_hbm, c6_hbm, out_hbm,
               x_v, y_v, z_v, idx_v, bit_v, word_v, out_v, c6_v, tab_s, sem):
        wid = lax.axis_index("s") * _NC + lax.axis_index("c")
        @pl.when(lax.axis_index("s") == 0)
        def _stage():
            pltpu.sync_copy(tab_hbm, tab_s)

        plsc.subcore_barrier()
        pltpu.sync_copy(c6_hbm, c6_v)
        sx = c6_v[pl.ds(0 * _L, _L)]
        sy = c6_v[pl.ds(1 * _L, _L)]
        sz = c6_v[pl.ds(2 * _L, _L)]
        tx = c6_v[pl.ds(3 * _L, _L)]
        ty = c6_v[pl.ds(4 * _L, _L)]
        tz = c6_v[pl.ds(5 * _L, _L)]
        hi_x = jnp.float32(d0) - 0.5
        hi_y = jnp.float32(d1) - 0.5
        hi_z = jnp.float32(d2) - 0.5

        def rnd(v, hi):
            c = jnp.clip(v, 0.0, hi - 0.5)
            f = (c + 0.5).astype(jnp.int32)
            tie = (f.astype(jnp.float32) - c) == 0.5
            return f - jnp.where(tie, f & 1, 0)

        def chunk_body(ci, base):
            base = pl.multiple_of(base, chunk)
            pltpu.sync_copy(x_hbm.at[pl.ds(base, chunk)], x_v)
            pltpu.sync_copy(y_hbm.at[pl.ds(base, chunk)], y_v)
            pltpu.sync_copy(z_hbm.at[pl.ds(base, chunk)], z_v)

            def pt(t, _):
                s = pl.ds(t * _L, _L)
                vx = x_v[s] * sx + tx
                vy = y_v[s] * sy + ty
                vz = z_v[s] * sz + tz
                inb = ((vx >= -0.5) & (vx < hi_x)
                       & (vy >= -0.5) & (vy < hi_y)
                       & (vz >= -0.5) & (vz < hi_z))
                fx = rnd(vx, hi_x)
                fy = rnd(vy, hi_y)
                fz = rnd(vz, hi_z)
                wi = (lax.shift_right_logical(fx, 5) * (d1 * d2)
                      + fy * d2) + fz
                idx_v[s] = jnp.where(inb, wi, oob)
                bit_v[s] = fx & 31
                return 0

            lax.fori_loop(0, chunk // _L, pt, 0)
            pltpu.async_copy(tab_s.at[idx_v], word_v, sem).wait()

            def ob(t, _):
                s = pl.ds(t * _L, _L)
                out_v[s] = lax.shift_right_logical(word_v[s], bit_v[s]) & 1
                return 0

            lax.fori_loop(0, chunk // _L, ob, 0)
            pltpu.sync_copy(out_v, out_hbm.at[pl.ds(base, chunk)])
            return base + chunk

        lax.fori_loop(0, n_chunks, chunk_body, wid * per_w)

    return lookup


def kernel(xyz, mask, xyz2ijk_scale, xyz2ijk_shift):
    shape = xyz.shape[:-1]
    n_pts = xyz.size // 3
    d0, d1, d2 = mask.shape
    m4 = mask.reshape(d0 // 32, 32, d1, d2).astype(jnp.uint32)
    bits = jnp.arange(32, dtype=jnp.uint32).reshape(1, 32, 1, 1)
    words3 = jnp.sum(m4 << bits, axis=1, dtype=jnp.uint32)
    tab = jnp.concatenate(
        [lax.bitcast_convert_type(words3, jnp.int32).reshape(-1),
         jnp.zeros((8,), jnp.int32)])
    xt = xyz.reshape(-1, 3).T
    c6 = jnp.repeat(
        jnp.concatenate([xyz2ijk_scale, xyz2ijk_shift])[:, None], _L, axis=1
    ).reshape(-1)
    lookup = _make_lookup(n_pts, d0, d1, d2, chunk=8192)
    out = lookup(xt[0], xt[1], xt[2], tab, c6)
    return out.astype(jnp.bool_).reshape(shape)


# speedup vs baseline: 14.4729x; 1.0562x over previous
import functools

import jax
import jax.numpy as jnp
from jax import lax
from jax.experimental import pallas as pl
from jax.experimental.pallas import tpu as pltpu
from jax.experimental.pallas import tpu_sc as plsc

_NC = 2
_NS = 16
_NW = _NC * _NS
_L = 16


def _make_lookup(n_pts, d0, d1, d2, chunk):
    n_words = (d0 * d1 * d2) // 32
    oob = n_words
    per_w = n_pts // _NW
    n_chunks = per_w // chunk
    mesh = plsc.VectorSubcoreMesh(core_axis_name="c", subcore_axis_name="s")

    @functools.partial(
        pl.kernel,
        mesh=mesh,
        out_type=jax.ShapeDtypeStruct((n_pts,), jnp.int32),
        scratch_types=[
            pltpu.VMEM((chunk,), jnp.float32),
            pltpu.VMEM((chunk,), jnp.float32),
            pltpu.VMEM((chunk,), jnp.float32),
            pltpu.VMEM((chunk,), jnp.int32),
            pltpu.VMEM((chunk,), jnp.int32),
            pltpu.VMEM((chunk,), jnp.int32),
            pltpu.VMEM((chunk,), jnp.int32),
            pltpu.VMEM((6 * _L,), jnp.float32),
            pltpu.VMEM_SHARED((n_words + 8,), jnp.int32),
            pltpu.SemaphoreType.DMA,
        ],
        compiler_params=pltpu.CompilerParams(needs_layout_passes=False),
    )
    def lookup(x_hbm, y_hbm, z_hbm, tab_hbm, c6_hbm, out_hbm,
               x_v, y_v, z_v, idx_v, bit_v, word_v, out_v, c6_v, tab_s, sem):
        wid = lax.axis_index("s") * _NC + lax.axis_index("c")
        @pl.when(lax.axis_index("s") == 0)
        def _stage():
            pltpu.sync_copy(tab_hbm, tab_s)

        plsc.subcore_barrier()
        pltpu.sync_copy(c6_hbm, c6_v)
        sx = c6_v[pl.ds(0 * _L, _L)]
        sy = c6_v[pl.ds(1 * _L, _L)]
        sz = c6_v[pl.ds(2 * _L, _L)]
        tx = c6_v[pl.ds(3 * _L, _L)]
        ty = c6_v[pl.ds(4 * _L, _L)]
        tz = c6_v[pl.ds(5 * _L, _L)]
        hi_x = jnp.float32(d0) - 0.5
        hi_y = jnp.float32(d1) - 0.5
        hi_z = jnp.float32(d2) - 0.5

        def rnd(v, hi):
            c = jnp.clip(v, 0.0, hi - 0.5)
            f = (c + 0.5).astype(jnp.int32)
            tie = (f.astype(jnp.float32) - c) == 0.5
            return f - jnp.where(tie, f & 1, 0)

        def chunk_body(ci, base):
            base = pl.multiple_of(base, chunk)
            pltpu.sync_copy(x_hbm.at[pl.ds(base, chunk)], x_v)
            pltpu.sync_copy(y_hbm.at[pl.ds(base, chunk)], y_v)
            pltpu.sync_copy(z_hbm.at[pl.ds(base, chunk)], z_v)

            @plsc.parallel_loop(0, chunk // _L, unroll=4)
            def pt(t):
                s = pl.ds(t * _L, _L)
                vx = x_v[s] * sx + tx
                vy = y_v[s] * sy + ty
                vz = z_v[s] * sz + tz
                inb = ((vx >= -0.5) & (vx < hi_x)
                       & (vy >= -0.5) & (vy < hi_y)
                       & (vz >= -0.5) & (vz < hi_z))
                fx = rnd(vx, hi_x)
                fy = rnd(vy, hi_y)
                fz = rnd(vz, hi_z)
                wi = (lax.shift_right_logical(fx, 5) * (d1 * d2)
                      + fy * d2) + fz
                idx_v[s] = jnp.where(inb, wi, oob)
                bit_v[s] = fx & 31

            pltpu.async_copy(tab_s.at[idx_v], word_v, sem).wait()

            @plsc.parallel_loop(0, chunk // _L, unroll=4)
            def ob(t):
                s = pl.ds(t * _L, _L)
                out_v[s] = lax.shift_right_logical(word_v[s], bit_v[s]) & 1
            pltpu.sync_copy(out_v, out_hbm.at[pl.ds(base, chunk)])
            return base + chunk

        lax.fori_loop(0, n_chunks, chunk_body, wid * per_w)

    return lookup


def kernel(xyz, mask, xyz2ijk_scale, xyz2ijk_shift):
    shape = xyz.shape[:-1]
    n_pts = xyz.size // 3
    d0, d1, d2 = mask.shape
    m4 = mask.reshape(d0 // 32, 32, d1, d2).astype(jnp.uint32)
    bits = jnp.arange(32, dtype=jnp.uint32).reshape(1, 32, 1, 1)
    words3 = jnp.sum(m4 << bits, axis=1, dtype=jnp.uint32)
    tab = jnp.concatenate(
        [lax.bitcast_convert_type(words3, jnp.int32).reshape(-1),
         jnp.zeros((8,), jnp.int32)])
    xt = xyz.reshape(-1, 3).T
    c6 = jnp.repeat(
        jnp.concatenate([xyz2ijk_scale, xyz2ijk_shift])[:, None], _L, axis=1
    ).reshape(-1)
    lookup = _make_lookup(n_pts, d0, d1, d2, chunk=8192)
    out = lookup(xt[0], xt[1], xt[2], tab, c6)
    return out.astype(jnp.bool_).reshape(shape)


# speedup vs baseline: 15.4429x; 1.0670x over previous
import functools

import jax
import jax.numpy as jnp
from jax import lax
from jax.experimental import pallas as pl
from jax.experimental.pallas import tpu as pltpu
from jax.experimental.pallas import tpu_sc as plsc

_NC = 2
_NS = 16
_NW = _NC * _NS
_L = 16


def _make_lookup(n_pts, d0, d1, d2, chunk, sub):
    n_words = (d0 * d1 * d2) // 32
    oob = n_words
    per_w = n_pts // _NW
    n_chunks = per_w // chunk
    n_sub = chunk // sub
    mesh = plsc.VectorSubcoreMesh(core_axis_name="c", subcore_axis_name="s")

    @functools.partial(
        pl.kernel,
        mesh=mesh,
        out_type=jax.ShapeDtypeStruct((n_pts,), jnp.int32),
        scratch_types=[
            pltpu.VMEM((chunk,), jnp.float32),
            pltpu.VMEM((chunk,), jnp.float32),
            pltpu.VMEM((chunk,), jnp.float32),
            pltpu.VMEM((sub,), jnp.int32),
            pltpu.VMEM((sub,), jnp.int32),
            pltpu.VMEM((chunk,), jnp.int32),
            pltpu.VMEM((sub,), jnp.int32),
            pltpu.VMEM((sub,), jnp.int32),
            pltpu.VMEM((chunk,), jnp.int32),
            pltpu.VMEM((6 * _L,), jnp.float32),
            pltpu.VMEM_SHARED((n_words + 8,), jnp.int32),
            pltpu.SemaphoreType.DMA,
            pltpu.SemaphoreType.DMA,
        ],
        compiler_params=pltpu.CompilerParams(needs_layout_passes=False),
    )
    def lookup(x_hbm, y_hbm, z_hbm, tab_hbm, c6_hbm, out_hbm,
               x_v, y_v, z_v, idx0_v, idx1_v, bit_v, word0_v, word1_v,
               out_v, c6_v, tab_s, sem0, sem1):
        wid = lax.axis_index("s") * _NC + lax.axis_index("c")
        @pl.when(lax.axis_index("s") == 0)
        def _stage():
            pltpu.sync_copy(tab_hbm, tab_s)

        plsc.subcore_barrier()
        pltpu.sync_copy(c6_hbm, c6_v)
        sx = c6_v[pl.ds(0 * _L, _L)]
        sy = c6_v[pl.ds(1 * _L, _L)]
        sz = c6_v[pl.ds(2 * _L, _L)]
        tx = c6_v[pl.ds(3 * _L, _L)]
        ty = c6_v[pl.ds(4 * _L, _L)]
        tz = c6_v[pl.ds(5 * _L, _L)]
        hi_x = jnp.float32(d0) - 0.5
        hi_y = jnp.float32(d1) - 0.5
        hi_z = jnp.float32(d2) - 0.5

        def rnd(v, hi):
            c = jnp.clip(v, 0.0, hi - 0.5)
            f = (c + 0.5).astype(jnp.int32)
            tie = (f.astype(jnp.float32) - c) == 0.5
            return f - jnp.where(tie, f & 1, 0)

        idx_ring = (idx0_v, idx1_v)
        word_ring = (word0_v, word1_v)
        sem_ring = (sem0, sem1)

        def pt(si):
            idx_r = idx_ring[si % 2]
            lo = si * sub

            @plsc.parallel_loop(0, sub // _L, unroll=4)
            def _(t):
                s = pl.ds(lo + t * _L, _L)
                vx = x_v[s] * sx + tx
                vy = y_v[s] * sy + ty
                vz = z_v[s] * sz + tz
                inb = ((vx >= -0.5) & (vx < hi_x)
                       & (vy >= -0.5) & (vy < hi_y)
                       & (vz >= -0.5) & (vz < hi_z))
                fx = rnd(vx, hi_x)
                fy = rnd(vy, hi_y)
                fz = rnd(vz, hi_z)
                wi = (lax.shift_right_logical(fx, 5) * (d1 * d2)
                      + fy * d2) + fz
                idx_r[pl.ds(t * _L, _L)] = jnp.where(inb, wi, oob)
                bit_v[s] = fx & 31

        def ob(si):
            word_r = word_ring[si % 2]
            lo = si * sub

            @plsc.parallel_loop(0, sub // _L, unroll=4)
            def _(t):
                s = pl.ds(lo + t * _L, _L)
                out_v[s] = (
                    lax.shift_right_logical(word_r[pl.ds(t * _L, _L)],
                                            bit_v[s]) & 1)

        def chunk_body(ci, base):
            base = pl.multiple_of(base, chunk)
            pltpu.sync_copy(x_hbm.at[pl.ds(base, chunk)], x_v)
            pltpu.sync_copy(y_hbm.at[pl.ds(base, chunk)], y_v)
            pltpu.sync_copy(z_hbm.at[pl.ds(base, chunk)], z_v)

            handles = [None, None]
            for si in range(n_sub):
                p = si % 2
                if handles[p] is not None:
                    handles[p].wait()
                    ob(si - 2)
                pt(si)
                handles[p] = pltpu.async_copy(
                    tab_s.at[idx_ring[p]], word_ring[p], sem_ring[p])
            for si in range(n_sub - 2, n_sub):
                handles[si % 2].wait()
                ob(si)

            pltpu.sync_copy(out_v, out_hbm.at[pl.ds(base, chunk)])
            return base + chunk

        lax.fori_loop(0, n_chunks, chunk_body, wid * per_w)

    return lookup


def kernel(xyz, mask, xyz2ijk_scale, xyz2ijk_shift):
    shape = xyz.shape[:-1]
    n_pts = xyz.size // 3
    d0, d1, d2 = mask.shape
    m4 = mask.reshape(d0 // 32, 32, d1, d2).astype(jnp.uint32)
    bits = jnp.arange(32, dtype=jnp.uint32).reshape(1, 32, 1, 1)
    words3 = jnp.sum(m4 << bits, axis=1, dtype=jnp.uint32)
    tab = jnp.concatenate(
        [lax.bitcast_convert_type(words3, jnp.int32).reshape(-1),
         jnp.zeros((8,), jnp.int32)])
    xt = xyz.reshape(-1, 3).T
    c6 = jnp.repeat(
        jnp.concatenate([xyz2ijk_scale, xyz2ijk_shift])[:, None], _L, axis=1
    ).reshape(-1)
    lookup = _make_lookup(n_pts, d0, d1, d2, chunk=8192, sub=2048)
    out = lookup(xt[0], xt[1], xt[2], tab, c6)
    return out.astype(jnp.bool_).reshape(shape)


# speedup vs baseline: 16.3364x; 1.0579x over previous
import functools

import jax
import jax.numpy as jnp
from jax import lax
from jax.experimental import pallas as pl
from jax.experimental.pallas import tpu as pltpu
from jax.experimental.pallas import tpu_sc as plsc

_NC = 2
_NS = 16
_NW = _NC * _NS
_L = 16


def _make_lookup(n_pts, d0, d1, d2, chunk, sub):
    n_words = (d0 * d1 * d2) // 32
    oob = n_words
    per_w = n_pts // _NW
    n_chunks = per_w // chunk
    n_sub = chunk // sub
    mesh = plsc.VectorSubcoreMesh(core_axis_name="c", subcore_axis_name="s")

    @functools.partial(
        pl.kernel,
        mesh=mesh,
        out_type=jax.ShapeDtypeStruct((n_pts,), jnp.int32),
        scratch_types=[
            pltpu.VMEM((chunk,), jnp.float32),
            pltpu.VMEM((chunk,), jnp.float32),
            pltpu.VMEM((chunk,), jnp.float32),
            pltpu.VMEM((sub,), jnp.int32),
            pltpu.VMEM((sub,), jnp.int32),
            pltpu.VMEM((chunk,), jnp.int32),
            pltpu.VMEM((sub,), jnp.int32),
            pltpu.VMEM((sub,), jnp.int32),
            pltpu.VMEM((chunk,), jnp.int32),
            pltpu.VMEM((6 * _L,), jnp.float32),
            pltpu.VMEM_SHARED((n_words + 8,), jnp.int32),
            pltpu.SemaphoreType.DMA,
            pltpu.SemaphoreType.DMA,
        ],
        compiler_params=pltpu.CompilerParams(needs_layout_passes=False),
    )
    def lookup(x_hbm, y_hbm, z_hbm, tab_hbm, c6_hbm, out_hbm,
               x_v, y_v, z_v, idx0_v, idx1_v, bit_v, word0_v, word1_v,
               out_v, c6_v, tab_s, sem0, sem1):
        wid = lax.axis_index("s") * _NC + lax.axis_index("c")
        @pl.when(lax.axis_index("s") == 0)
        def _stage():
            pltpu.sync_copy(tab_hbm, tab_s)

        plsc.subcore_barrier()
        pltpu.sync_copy(c6_hbm, c6_v)
        sx = c6_v[pl.ds(0 * _L, _L)]
        sy = c6_v[pl.ds(1 * _L, _L)]
        sz = c6_v[pl.ds(2 * _L, _L)]
        tx = c6_v[pl.ds(3 * _L, _L)]
        ty = c6_v[pl.ds(4 * _L, _L)]
        tz = c6_v[pl.ds(5 * _L, _L)]
        hi_x = jnp.float32(d0) - 0.5
        hi_y = jnp.float32(d1) - 0.5
        hi_z = jnp.float32(d2) - 0.5

        def rnd(v, hi):
            c = jnp.clip(v, 0.0, hi - 0.5)
            f = (c + 0.5).astype(jnp.int32)
            tie = (f.astype(jnp.float32) - c) == 0.5
            return f - jnp.where(tie, f & 1, 0)

        idx_ring = (idx0_v, idx1_v)
        word_ring = (word0_v, word1_v)
        sem_ring = (sem0, sem1)

        def pt(si):
            idx_r = idx_ring[si % 2]
            lo = si * sub

            @plsc.parallel_loop(0, sub // _L, unroll=4)
            def _(t):
                s = pl.ds(lo + t * _L, _L)
                vx = x_v[s] * sx + tx
                vy = y_v[s] * sy + ty
                vz = z_v[s] * sz + tz
                inb = ((vx >= -0.5) & (vx < hi_x)
                       & (vy >= -0.5) & (vy < hi_y)
                       & (vz >= -0.5) & (vz < hi_z))
                fx = rnd(vx, hi_x)
                fy = rnd(vy, hi_y)
                fz = rnd(vz, hi_z)
                wi = (lax.shift_right_logical(fx, 5) * (d1 * d2)
                      + fy * d2) + fz
                idx_r[pl.ds(t * _L, _L)] = jnp.where(inb, wi, oob)
                bit_v[s] = fx & 31

        def ob(si):
            word_r = word_ring[si % 2]
            lo = si * sub

            @plsc.parallel_loop(0, sub // _L, unroll=4)
            def _(t):
                s = pl.ds(lo + t * _L, _L)
                out_v[s] = (
                    lax.shift_right_logical(word_r[pl.ds(t * _L, _L)],
                                            bit_v[s]) & 1)

        def chunk_body(ci, base):
            base = pl.multiple_of(base, chunk)
            hx = pltpu.async_copy(x_hbm.at[pl.ds(base, chunk)], x_v, sem0)
            hy = pltpu.async_copy(y_hbm.at[pl.ds(base, chunk)], y_v, sem0)
            hz = pltpu.async_copy(z_hbm.at[pl.ds(base, chunk)], z_v, sem0)
            hx.wait()
            hy.wait()
            hz.wait()

            handles = [None, None]
            for si in range(n_sub):
                p = si % 2
                if handles[p] is not None:
                    handles[p].wait()
                    ob(si - 2)
                pt(si)
                handles[p] = pltpu.async_copy(
                    tab_s.at[idx_ring[p]], word_ring[p], sem_ring[p])
            for si in range(n_sub - 2, n_sub):
                handles[si % 2].wait()
                ob(si)

            pltpu.sync_copy(out_v, out_hbm.at[pl.ds(base, chunk)])
            return base + chunk

        lax.fori_loop(0, n_chunks, chunk_body, wid * per_w)

    return lookup


def kernel(xyz, mask, xyz2ijk_scale, xyz2ijk_shift):
    shape = xyz.shape[:-1]
    n_pts = xyz.size // 3
    d0, d1, d2 = mask.shape
    m4 = mask.reshape(d0 // 32, 32, d1, d2).astype(jnp.uint32)
    bits = jnp.arange(32, dtype=jnp.uint32).reshape(1, 32, 1, 1)
    words3 = jnp.sum(m4 << bits, axis=1, dtype=jnp.uint32)
    tab = jnp.concatenate(
        [lax.bitcast_convert_type(words3, jnp.int32).reshape(-1),
         jnp.zeros((8,), jnp.int32)])
    xt = xyz.reshape(-1, 3).T
    c6 = jnp.repeat(
        jnp.concatenate([xyz2ijk_scale, xyz2ijk_shift])[:, None], _L, axis=1
    ).reshape(-1)
    lookup = _make_lookup(n_pts, d0, d1, d2, chunk=8192, sub=2048)
    out = lookup(xt[0], xt[1], xt[2], tab, c6)
    return out.astype(jnp.bool_).reshape(shape)


# speedup vs baseline: 16.9276x; 1.0362x over previous
import functools

import jax
import jax.numpy as jnp
from jax import lax
from jax.experimental import pallas as pl
from jax.experimental.pallas import tpu as pltpu
from jax.experimental.pallas import tpu_sc as plsc

_NC = 2
_NS = 16
_NW = _NC * _NS
_L = 16


def _make_lookup(n_pts, d0, d1, d2, chunk, sub):
    n_words = (d0 * d1 * d2) // 32
    oob = n_words
    per_w = n_pts // _NW
    n_chunks = per_w // chunk
    n_sub = chunk // sub
    mesh = plsc.VectorSubcoreMesh(core_axis_name="c", subcore_axis_name="s")

    @functools.partial(
        pl.kernel,
        mesh=mesh,
        out_type=jax.ShapeDtypeStruct((n_pts,), jnp.int32),
        scratch_types=[
            pltpu.VMEM((2, chunk), jnp.float32),
            pltpu.VMEM((2, chunk), jnp.float32),
            pltpu.VMEM((2, chunk), jnp.float32),
            pltpu.VMEM((sub,), jnp.int32),
            pltpu.VMEM((sub,), jnp.int32),
            pltpu.VMEM((chunk,), jnp.int32),
            pltpu.VMEM((sub,), jnp.int32),
            pltpu.VMEM((sub,), jnp.int32),
            pltpu.VMEM((chunk,), jnp.int32),
            pltpu.VMEM((6 * _L,), jnp.float32),
            pltpu.VMEM_SHARED((n_words + 8,), jnp.int32),
            pltpu.SemaphoreType.DMA,
            pltpu.SemaphoreType.DMA,
            pltpu.SemaphoreType.DMA,
        ],
        compiler_params=pltpu.CompilerParams(needs_layout_passes=False),
    )
    def lookup(x_hbm, y_hbm, z_hbm, tab_hbm, c6_hbm, out_hbm,
               x_v, y_v, z_v, idx0_v, idx1_v, bit_v, word0_v, word1_v,
               out_v, c6_v, tab_s, sem0, sem1, semx):
        wid = lax.axis_index("s") * _NC + lax.axis_index("c")
        @pl.when(lax.axis_index("s") == 0)
        def _stage():
            pltpu.sync_copy(tab_hbm, tab_s)

        plsc.subcore_barrier()
        pltpu.sync_copy(c6_hbm, c6_v)
        sx = c6_v[pl.ds(0 * _L, _L)]
        sy = c6_v[pl.ds(1 * _L, _L)]
        sz = c6_v[pl.ds(2 * _L, _L)]
        tx = c6_v[pl.ds(3 * _L, _L)]
        ty = c6_v[pl.ds(4 * _L, _L)]
        tz = c6_v[pl.ds(5 * _L, _L)]
        hi_x = jnp.float32(d0) - 0.5
        hi_y = jnp.float32(d1) - 0.5
        hi_z = jnp.float32(d2) - 0.5

        def rnd(v, hi):
            c = jnp.clip(v, 0.0, hi - 0.5)
            f = (c + 0.5).astype(jnp.int32)
            tie = (f.astype(jnp.float32) - c) == 0.5
            return f - jnp.where(tie, f & 1, 0)

        idx_ring = (idx0_v, idx1_v)
        word_ring = (word0_v, word1_v)
        sem_ring = (sem0, sem1)
        cube = d0 == d1 and d1 == d2

        def pt(buf, si):
            idx_r = idx_ring[si % 2]
            lo = si * sub

            @plsc.parallel_loop(0, sub // _L, unroll=4)
            def _(t):
                s = pl.ds(lo + t * _L, _L)
                vx = x_v[buf, s] * sx + tx
                vy = y_v[buf, s] * sy + ty
                vz = z_v[buf, s] * sz + tz
                if cube:
                    lo3 = jnp.minimum(jnp.minimum(vx, vy), vz)
                    hi3 = jnp.maximum(jnp.maximum(vx, vy), vz)
                    inb = (lo3 >= -0.5) & (hi3 < hi_x)
                else:
                    inb = ((vx >= -0.5) & (vx < hi_x)
                           & (vy >= -0.5) & (vy < hi_y)
                           & (vz >= -0.5) & (vz < hi_z))
                fx = rnd(vx, hi_x)
                fy = rnd(vy, hi_y)
                fz = rnd(vz, hi_z)
                wi = (lax.shift_right_logical(fx, 5) * (d1 * d2)
                      + fy * d2) + fz
                idx_r[pl.ds(t * _L, _L)] = jnp.where(inb, wi, oob)
                bit_v[s] = fx & 31

        def ob(si):
            word_r = word_ring[si % 2]
            lo = si * sub

            @plsc.parallel_loop(0, sub // _L, unroll=4)
            def _(t):
                s = pl.ds(lo + t * _L, _L)
                out_v[s] = (
                    lax.shift_right_logical(word_r[pl.ds(t * _L, _L)],
                                            bit_v[s]) & 1)

        def issue_xyz(buf, base):
            pltpu.async_copy(x_hbm.at[pl.ds(base, chunk)], x_v.at[buf], semx)
            pltpu.async_copy(y_hbm.at[pl.ds(base, chunk)], y_v.at[buf], semx)
            pltpu.async_copy(z_hbm.at[pl.ds(base, chunk)], z_v.at[buf], semx)

        def wait_xyz(buf):
            pltpu.make_async_copy(
                x_hbm.at[pl.ds(0, chunk)], x_v.at[buf], semx).wait()
            pltpu.make_async_copy(
                y_hbm.at[pl.ds(0, chunk)], y_v.at[buf], semx).wait()
            pltpu.make_async_copy(
                z_hbm.at[pl.ds(0, chunk)], z_v.at[buf], semx).wait()

        def process(buf, base):
            handles = [None, None]
            for si in range(n_sub):
                p = si % 2
                if handles[p] is not None:
                    handles[p].wait()
                    ob(si - 2)
                pt(buf, si)
                handles[p] = pltpu.async_copy(
                    tab_s.at[idx_ring[p]], word_ring[p], sem_ring[p])
            for si in range(n_sub - 2, n_sub):
                handles[si % 2].wait()
                ob(si)

            pltpu.sync_copy(out_v, out_hbm.at[pl.ds(base, chunk)])

        start = wid * per_w
        issue_xyz(0, pl.multiple_of(start, chunk))

        def pair_body(k, base):
            base = pl.multiple_of(base, chunk)
            wait_xyz(0)
            issue_xyz(1, base + chunk)
            process(0, base)
            wait_xyz(1)
            nxt = pl.multiple_of(
                jnp.minimum(base + 2 * chunk, n_pts - chunk), chunk)
            issue_xyz(0, nxt)
            process(1, base + chunk)
            return base + 2 * chunk

        lax.fori_loop(0, n_chunks // 2, pair_body, start)
        wait_xyz(0)

    return lookup


def kernel(xyz, mask, xyz2ijk_scale, xyz2ijk_shift):
    shape = xyz.shape[:-1]
    n_pts = xyz.size // 3
    d0, d1, d2 = mask.shape
    m4 = mask.reshape(d0 // 32, 32, d1, d2).astype(jnp.uint32)
    bits = jnp.arange(32, dtype=jnp.uint32).reshape(1, 32, 1, 1)
    words3 = jnp.sum(m4 << bits, axis=1, dtype=jnp.uint32)
    tab = jnp.concatenate(
        [lax.bitcast_convert_type(words3, jnp.int32).reshape(-1),
         jnp.zeros((8,), jnp.int32)])
    xt = xyz.reshape(-1, 3).T
    c6 = jnp.repeat(
        jnp.concatenate([xyz2ijk_scale, xyz2ijk_shift])[:, None], _L, axis=1
    ).reshape(-1)
    lookup = _make_lookup(n_pts, d0, d1, d2, chunk=8192, sub=2048)
    out = lookup(xt[0], xt[1], xt[2], tab, c6)
    return out.astype(jnp.bool_).reshape(shape)
